# baseline (speedup 1.0000x reference)
import jax
import jax.numpy as jnp
from jax import lax
from jax.experimental import pallas as pl
from jax.experimental.pallas import tpu as pltpu

B, N, D, C, IN, S = 64, 8192, 64, 256, 128, 3
CTRL_IN = IN + D
EPS = 1e-8

N2 = N // 2
BB = 8
NJ = 512
GB = B // BB
GJ = N2 // NJ

_DN = (((1,), (1,)), ((), ()))
_DNK = (((1,), (0,)), ((), ()))


def _prologue_kernel(ctrl_in_ref, W_ctrl_ref, b_ctrl_ref,
                     Wk_r_ref, bk_r_ref, Wk_w_ref, bk_w_ref,
                     We_w_ref, be_w_ref, Wa_w_ref, ba_w_ref,
                     Wsc_r_ref, bsc_r_ref, Wsc_w_ref, bsc_w_ref,
                     pw3_ref, pr3_ref,
                     h_ref, erase_ref, add_ref, Lm_ref, Lsq_ref,
                     par_r_ref, par_w_ref,
                     pwe_ref, pwo_ref, pre_ref, pro_ref):
    f32 = jnp.float32
    t = lax.broadcasted_iota(jnp.int32, (B, 64, 128), 2)
    idx = 2 * (t % 64) + (t >= 64).astype(jnp.int32)
    for src_ref, eref, oref in ((pw3_ref, pwe_ref, pwo_ref),
                                (pr3_ref, pre_ref, pro_ref)):
        xp = jnp.take_along_axis(src_ref[...], idx, axis=2)
        eref[...] = xp[:, :, :64]
        oref[...] = xp[:, :, 64:]
    h = jnp.maximum(
        jnp.dot(ctrl_in_ref[...], W_ctrl_ref[...],
                preferred_element_type=f32) + b_ctrl_ref[...], 0.0)
    h_ref[...] = h
    k_r = jnp.dot(h, Wk_r_ref[...], preferred_element_type=f32) + bk_r_ref[...]
    k_w = jnp.dot(h, Wk_w_ref[...], preferred_element_type=f32) + bk_w_ref[...]
    e = jax.nn.sigmoid(
        jnp.dot(h, We_w_ref[...], preferred_element_type=f32) + be_w_ref[...])
    a = jnp.tanh(
        jnp.dot(h, Wa_w_ref[...], preferred_element_type=f32) + ba_w_ref[...])
    erase_ref[...] = e
    add_ref[...] = a

    z64 = jnp.zeros((B, 64), f32)
    ekr = e * k_r
    ae = a * e
    ones64 = jnp.ones((B, 64), f32)
    esq = e * e

    def pack_rows(rows):
        evens = [jnp.concatenate([v, z64], -1)[:, None, :] for v in rows]
        odds = [jnp.concatenate([z64, v], -1)[:, None, :] for v in rows]
        pad = jnp.zeros((B, 8 - len(rows), 128), f32)
        return jnp.concatenate(evens + [pad] + odds + [pad],
                               axis=1).astype(jnp.bfloat16)

    Lm_ref[...] = pack_rows([k_w, k_r, ekr, a, ae])
    Lsq_ref[...] = pack_rows([ones64, e, esq])

    ksq_r = jnp.sum(k_r * k_r, axis=-1, keepdims=True)
    ksq_w = jnp.sum(k_w * k_w, axis=-1, keepdims=True)
    ak = jnp.sum(a * k_r, axis=-1, keepdims=True)
    asq = jnp.sum(a * a, axis=-1, keepdims=True)
    for Wsc_ref, bsc_ref, ksq, extra, par_ref in (
            (Wsc_r_ref, bsc_r_ref, ksq_r, [ak, asq], par_r_ref),
            (Wsc_w_ref, bsc_w_ref, ksq_w, [], par_w_ref)):
        raw = (jnp.dot(h, Wsc_ref[...], preferred_element_type=f32)
               + bsc_ref[...])
        beta = jax.nn.softplus(raw[:, 0:1])
        g = jax.nn.sigmoid(raw[:, 1:2])
        gamma = jax.nn.softplus(raw[:, 2:3]) + 1.0
        slog = raw[:, 3:6]
        smax = jnp.max(slog, axis=-1, keepdims=True)
        sexp = jnp.exp(slog - smax)
        s = sexp / jnp.sum(sexp, axis=-1, keepdims=True)
        cols = [beta, g, gamma, s, ksq] + extra
        used = 7 + len(extra)
        cols.append(jnp.zeros((B, 128 - used), f32))
        par_ref[...] = jnp.concatenate(cols, axis=-1)


def _run_prologue(ctrl_in, W_ctrl, b_ctrl, Wk_r, bk_r, Wk_w, bk_w,
                  We_w, be_w, Wa_w, ba_w, Wsc_r, bsc_r, Wsc_w, bsc_w,
                  pw3, pr3):
    plane = jax.ShapeDtypeStruct((B, 64, 64), jnp.float32)
    out_shapes = (
        jax.ShapeDtypeStruct((B, C), jnp.float32),
        jax.ShapeDtypeStruct((B, D), jnp.float32),
        jax.ShapeDtypeStruct((B, D), jnp.float32),
        jax.ShapeDtypeStruct((B, 16, 128), jnp.bfloat16),
        jax.ShapeDtypeStruct((B, 16, 128), jnp.bfloat16),
        jax.ShapeDtypeStruct((B, 128), jnp.float32),
        jax.ShapeDtypeStruct((B, 128), jnp.float32),
        plane, plane, plane, plane,
    )
    return pl.pallas_call(
        _prologue_kernel,
        out_shape=out_shapes,
    )(ctrl_in, W_ctrl, b_ctrl, Wk_r, bk_r, Wk_w, bk_w,
      We_w, be_w, Wa_w, ba_w, Wsc_r, bsc_r, Wsc_w, bsc_w, pw3, pr3)


def _pass1_kernel(mem_ref, Lm_ref, Lsq_ref,
                  dkw_ref, dkr_ref, dekr_ref, da_ref, dae_ref,
                  ssq_ref, ssqe_ref, ssqee_ref):
    f32 = jnp.float32
    for b in range(BB):
        mb = mem_ref[b].astype(jnp.bfloat16)
        sq = mb * mb
        om = lax.dot_general(Lm_ref[b], mb, _DN,
                             preferred_element_type=f32)
        osq = lax.dot_general(Lsq_ref[b], sq, _DN,
                              preferred_element_type=f32)
        for r, ref in enumerate((dkw_ref, dkr_ref, dekr_ref, da_ref, dae_ref)):
            ref[0, b, :] = om[r]
            ref[1, b, :] = om[r + 8]
        for r, ref in enumerate((ssq_ref, ssqe_ref, ssqee_ref)):
            ref[0, b, :] = osq[r]
            ref[1, b, :] = osq[r + 8]


def _run_pass1(mem2, Lm, Lsq):
    big = pl.BlockSpec((BB, NJ, 128), lambda i, j: (i, j, 0))
    lspec = pl.BlockSpec((BB, 16, 128), lambda i, j: (i, 0, 0))
    ospec = pl.BlockSpec((2, BB, NJ), lambda i, j: (0, i, j))
    oshape = jax.ShapeDtypeStruct((2, B, N2), jnp.float32)
    return pl.pallas_call(
        _pass1_kernel,
        grid=(GB, GJ),
        in_specs=[big, lspec, lspec],
        out_specs=[ospec] * 8,
        out_shape=[oshape] * 8,
        compiler_params=pltpu.CompilerParams(
            dimension_semantics=("parallel", "arbitrary")),
    )(mem2, Lm, Lsq)


def _address(dots_e, dots_o, sqn_e, sqn_o, par, pw_e, pw_o):
    beta = par[:, 0:1]
    g = par[:, 1:2]
    gamma = par[:, 2:3]
    s0 = par[:, 3:4]
    s1 = par[:, 4:5]
    s2 = par[:, 5:6]
    knorm = jnp.sqrt(par[:, 6:7])

    def zfun(dots, sqn):
        norms = jnp.sqrt(jnp.maximum(sqn, 0.0)) * knorm
        return beta * (dots / (norms + EPS))

    z_e = zfun(dots_e, sqn_e)
    z_o = zfun(dots_o, sqn_o)
    zmax = jnp.maximum(jnp.max(z_e, axis=-1, keepdims=True),
                       jnp.max(z_o, axis=-1, keepdims=True))
    ex_e = jnp.exp(z_e - zmax)
    ex_o = jnp.exp(z_o - zmax)
    zsum = (jnp.sum(ex_e, axis=-1, keepdims=True)
            + jnp.sum(ex_o, axis=-1, keepdims=True))
    inv = 1.0 / zsum
    wg_e = g * (ex_e * inv) + (1.0 - g) * pw_e
    wg_o = g * (ex_o * inv) + (1.0 - g) * pw_o
    lsh_e = jnp.concatenate([wg_e[:, 1:], wg_e[:, :1]], axis=-1)
    rsh_o = jnp.concatenate([wg_o[:, -1:], wg_o[:, :-1]], axis=-1)
    ws_e = s0 * wg_o + s1 * wg_e + s2 * rsh_o
    ws_o = s0 * lsh_e + s1 * wg_o + s2 * wg_e
    u_e = jnp.exp(gamma * jnp.log(ws_e))
    u_o = jnp.exp(gamma * jnp.log(ws_o))
    usum = (jnp.sum(u_e, axis=-1, keepdims=True)
            + jnp.sum(u_o, axis=-1, keepdims=True))
    uinv = 1.0 / (usum + EPS)
    return u_e * uinv, u_o * uinv


def _fin_w_kernel(dkw_ref, ssq_ref, par_ref, pwe_ref, pwo_ref, ww_ref):
    w_e, w_o = _address(dkw_ref[0], dkw_ref[1], ssq_ref[0], ssq_ref[1],
                        par_ref[...], pwe_ref[...], pwo_ref[...])
    ww_ref[0] = w_e
    ww_ref[1] = w_o


def _run_fin_w(dkw, ssq, par_w, pwe, pwo):
    plane = pl.BlockSpec((2, BB, N2), lambda i: (0, i, 0))
    half = pl.BlockSpec((BB, N2), lambda i: (i, 0))
    return pl.pallas_call(
        _fin_w_kernel,
        grid=(GB,),
        in_specs=[plane, plane,
                  pl.BlockSpec((BB, 128), lambda i: (i, 0)), half, half],
        out_specs=plane,
        out_shape=jax.ShapeDtypeStruct((2, B, N2), jnp.float32),
        compiler_params=pltpu.CompilerParams(
            dimension_semantics=("parallel",)),
    )(dkw, ssq, par_w, pwe, pwo)


def _fin_r_kernel(dkr_ref, dekr_ref, da_ref, dae_ref,
                  ssq_ref, ssqe_ref, ssqee_ref,
                  ww_ref, par_ref, pre_ref, pro_ref,
                  wr_ref, wrww_ref, swr_ref):
    par = par_ref[...]
    ak = par[:, 7:8]
    asq = par[:, 8:9]
    dots = []
    sqns = []
    for h in range(2):
        ww = ww_ref[h]
        wwsq = ww * ww
        dots.append(dkr_ref[h] - ww * dekr_ref[h] + ww * ak)
        sqns.append(ssq_ref[h] - 2.0 * ww * ssqe_ref[h] + wwsq * ssqee_ref[h]
                    + 2.0 * ww * da_ref[h] - 2.0 * wwsq * dae_ref[h]
                    + wwsq * asq)
    w_e, w_o = _address(dots[0], dots[1], sqns[0], sqns[1],
                        par, pre_ref[...], pro_ref[...])
    wr_ref[0] = w_e
    wr_ref[1] = w_o
    p_e = w_e * ww_ref[0]
    p_o = w_o * ww_ref[1]
    wrww_ref[0] = p_e
    wrww_ref[1] = p_o
    swr_ref[...] = (jnp.sum(p_e, axis=-1, keepdims=True)
                    + jnp.sum(p_o, axis=-1, keepdims=True))


def _run_fin_r(dkr, dekr, da, dae, ssq, ssqe, ssqee, ww2, par_r, pre, pro):
    plane = pl.BlockSpec((2, BB, N2), lambda i: (0, i, 0))
    half = pl.BlockSpec((BB, N2), lambda i: (i, 0))
    return pl.pallas_call(
        _fin_r_kernel,
        grid=(GB,),
        in_specs=[plane] * 7 + [plane,
                                pl.BlockSpec((BB, 128), lambda i: (i, 0)),
                                half, half],
        out_specs=[plane, plane, pl.BlockSpec((BB, 1), lambda i: (i, 0))],
        out_shape=[jax.ShapeDtypeStruct((2, B, N2), jnp.float32),
                   jax.ShapeDtypeStruct((2, B, N2), jnp.float32),
                   jax.ShapeDtypeStruct((B, 1), jnp.float32)],
        compiler_params=pltpu.CompilerParams(
            dimension_semantics=("parallel",)),
    )(dkr, dekr, da, dae, ssq, ssqe, ssqee, ww2, par_r, pre, pro)


def _pass2_kernel(mem_ref, wr_ref, wrww_ref, erase_ref, add_ref, swr_ref,
                  out_ref, acc_ref):
    f32 = jnp.float32
    j = pl.program_id(1)

    @pl.when(j == 0)
    def _init():
        acc_ref[...] = jnp.zeros_like(acc_ref)

    for b in range(BB):
        mb = mem_ref[b].astype(jnp.bfloat16)
        lhs = jnp.concatenate(
            [wr_ref[0, b:b + 1, :], wr_ref[1, b:b + 1, :],
             wrww_ref[0, b:b + 1, :], wrww_ref[1, b:b + 1, :],
             jnp.zeros((4, NJ), f32)], axis=0).astype(jnp.bfloat16)
        o = lax.dot_general(lhs, mb, _DNK,
                            preferred_element_type=f32)
        acc_ref[b] += o

    @pl.when(j == GJ - 1)
    def _fin():
        acc = acc_ref[...]
        a0 = acc[:, 0, :]
        a1 = acc[:, 1, :]
        a2 = acc[:, 2, :]
        a3 = acc[:, 3, :]
        sh1 = jnp.concatenate([a1[:, 64:], a1[:, :64]], axis=-1)
        sh3 = jnp.concatenate([a3[:, 64:], a3[:, :64]], axis=-1)
        p1 = (a0 + sh1)[:, :64]
        p2 = (a2 + sh3)[:, :64]
        out_ref[...] = (p1 - erase_ref[...] * p2
                        + swr_ref[...] * add_ref[...])


def _run_pass2(mem2, wr2, wrww2, erase, add, swr):
    big = pl.BlockSpec((BB, NJ, 128), lambda i, j: (i, j, 0))
    wspec = pl.BlockSpec((2, BB, NJ), lambda i, j: (0, i, j))
    vec = pl.BlockSpec((BB, D), lambda i, j: (i, 0))
    return pl.pallas_call(
        _pass2_kernel,
        grid=(GB, GJ),
        in_specs=[big, wspec, wspec, vec, vec,
                  pl.BlockSpec((BB, 1), lambda i, j: (i, 0))],
        out_specs=vec,
        out_shape=jax.ShapeDtypeStruct((B, D), jnp.float32),
        scratch_shapes=[pltpu.VMEM((BB, 8, 128), jnp.float32)],
        compiler_params=pltpu.CompilerParams(
            dimension_semantics=("parallel", "arbitrary")),
    )(mem2, wr2, wrww2, erase, add, swr)


@jax.jit
def kernel(x, prev_memory, prev_read_weights, prev_write_weights,
           prev_read_vector,
           W_ctrl, b_ctrl,
           Wk_r, bk_r, Wb_r, bb_r, Wg_r, bg_r, Ws_r, bs_r, Wgam_r, bgam_r,
           Wk_w, bk_w, Wb_w, bb_w, Wg_w, bg_w, Ws_w, bs_w, Wgam_w, bgam_w,
           We_w, be_w, Wa_w, ba_w):
    ctrl_in = jnp.concatenate([x, prev_read_vector], axis=-1)

    def pack_scalar_weights(Wb, Wg, Wgam, Ws, bb, bg, bgam, bs):
        Wsc = jnp.concatenate([Wb, Wg, Wgam, Ws], axis=-1)
        Wsc = jnp.pad(Wsc, ((0, 0), (0, 122)))
        bsc = jnp.concatenate([bb, bg, bgam, bs], axis=-1)
        bsc = jnp.pad(bsc, (0, 122)).reshape(1, 128)
        return Wsc, bsc

    Wsc_r, bsc_r = pack_scalar_weights(Wb_r, Wg_r, Wgam_r, Ws_r,
                                       bb_r, bg_r, bgam_r, bs_r)
    Wsc_w, bsc_w = pack_scalar_weights(Wb_w, Wg_w, Wgam_w, Ws_w,
                                       bb_w, bg_w, bgam_w, bs_w)

    (h, erase, add, Lm, Lsq, par_r, par_w,
     pwe, pwo, pre, pro) = _run_prologue(
        ctrl_in, W_ctrl, b_ctrl.reshape(1, C),
        Wk_r, bk_r.reshape(1, D), Wk_w, bk_w.reshape(1, D),
        We_w, be_w.reshape(1, D), Wa_w, ba_w.reshape(1, D),
        Wsc_r, bsc_r, Wsc_w, bsc_w,
        prev_write_weights.reshape(B, 64, 128),
        prev_read_weights.reshape(B, 64, 128))

    mem2 = prev_memory.reshape(B, N2, 128)

    dkw, dkr, dekr, da, dae, ssq, ssqe, ssqee = _run_pass1(mem2, Lm, Lsq)
    ww2 = _run_fin_w(dkw, ssq, par_w,
                     pwe.reshape(B, N2), pwo.reshape(B, N2))
    wr2, wrww2, swr = _run_fin_r(dkr, dekr, da, dae, ssq, ssqe, ssqee,
                                 ww2, par_r,
                                 pre.reshape(B, N2), pro.reshape(B, N2))
    read_vec = _run_pass2(mem2, wr2, wrww2, erase, add, swr)
    return jnp.concatenate([h, read_vec], axis=-1)


# speedup vs baseline: 2.6621x; 2.6621x over previous
import jax
import jax.numpy as jnp
from jax import lax
from jax.experimental import pallas as pl
from jax.experimental.pallas import tpu as pltpu

B, N, D, C, IN, S = 64, 8192, 64, 256, 128, 3
CTRL_IN = IN + D
EPS = 1e-8

BB = 8
NB = 1024
GB = B // BB
GJ = N // NB

_DNK = (((1,), (0,)), ((), ()))


def _prologue_kernel(ctrl_in_ref, W_ctrl_ref, b_ctrl_ref,
                     Wk_r_ref, bk_r_ref, Wk_w_ref, bk_w_ref,
                     We_w_ref, be_w_ref, Wa_w_ref, ba_w_ref,
                     Wsc_r_ref, bsc_r_ref, Wsc_w_ref, bsc_w_ref,
                     h_ref, erase_ref, add_ref, Lm_ref, Lsq_ref,
                     par_r_ref, par_w_ref):
    f32 = jnp.float32
    h = jnp.maximum(
        jnp.dot(ctrl_in_ref[...], W_ctrl_ref[...],
                preferred_element_type=f32) + b_ctrl_ref[...], 0.0)
    h_ref[...] = h
    k_r = jnp.dot(h, Wk_r_ref[...], preferred_element_type=f32) + bk_r_ref[...]
    k_w = jnp.dot(h, Wk_w_ref[...], preferred_element_type=f32) + bk_w_ref[...]
    e = jax.nn.sigmoid(
        jnp.dot(h, We_w_ref[...], preferred_element_type=f32) + be_w_ref[...])
    a = jnp.tanh(
        jnp.dot(h, Wa_w_ref[...], preferred_element_type=f32) + ba_w_ref[...])
    erase_ref[...] = e
    add_ref[...] = a

    def pack_rows(rows):
        rs = [v[:, None, :] for v in rows]
        pad = jnp.zeros((B, 8 - len(rows), D), f32)
        return jnp.concatenate(rs + [pad], axis=1).astype(jnp.bfloat16)

    Lm_ref[...] = pack_rows([k_w, k_r, e * k_r, a, a * e])
    Lsq_ref[...] = pack_rows([jnp.ones((B, D), f32), e, e * e])

    ksq_r = jnp.sum(k_r * k_r, axis=-1, keepdims=True)
    ksq_w = jnp.sum(k_w * k_w, axis=-1, keepdims=True)
    ak = jnp.sum(a * k_r, axis=-1, keepdims=True)
    asq = jnp.sum(a * a, axis=-1, keepdims=True)
    for Wsc_ref, bsc_ref, ksq, extra, par_ref in (
            (Wsc_r_ref, bsc_r_ref, ksq_r, [ak, asq], par_r_ref),
            (Wsc_w_ref, bsc_w_ref, ksq_w, [], par_w_ref)):
        raw = (jnp.dot(h, Wsc_ref[...], preferred_element_type=f32)
               + bsc_ref[...])
        beta = jax.nn.softplus(raw[:, 0:1])
        g = jax.nn.sigmoid(raw[:, 1:2])
        gamma = jax.nn.softplus(raw[:, 2:3]) + 1.0
        slog = raw[:, 3:6]
        smax = jnp.max(slog, axis=-1, keepdims=True)
        sexp = jnp.exp(slog - smax)
        s = sexp / jnp.sum(sexp, axis=-1, keepdims=True)
        cols = [beta, g, gamma, s, ksq] + extra
        used = 7 + len(extra)
        cols.append(jnp.zeros((B, 128 - used), f32))
        par_ref[...] = jnp.concatenate(cols, axis=-1)


def _run_prologue(ctrl_in, W_ctrl, b_ctrl, Wk_r, bk_r, Wk_w, bk_w,
                  We_w, be_w, Wa_w, ba_w, Wsc_r, bsc_r, Wsc_w, bsc_w):
    out_shapes = (
        jax.ShapeDtypeStruct((B, C), jnp.float32),
        jax.ShapeDtypeStruct((B, D), jnp.float32),
        jax.ShapeDtypeStruct((B, D), jnp.float32),
        jax.ShapeDtypeStruct((B, 8, D), jnp.bfloat16),
        jax.ShapeDtypeStruct((B, 8, D), jnp.bfloat16),
        jax.ShapeDtypeStruct((B, 128), jnp.float32),
        jax.ShapeDtypeStruct((B, 128), jnp.float32),
    )
    return pl.pallas_call(
        _prologue_kernel,
        out_shape=out_shapes,
    )(ctrl_in, W_ctrl, b_ctrl, Wk_r, bk_r, Wk_w, bk_w,
      We_w, be_w, Wa_w, ba_w, Wsc_r, bsc_r, Wsc_w, bsc_w)


def _pass1_kernel(mem_ref, Lm_ref, Lsq_ref,
                  dkw_ref, dkr_ref, dekr_ref, da_ref, dae_ref,
                  ssq_ref, ssqe_ref, ssqee_ref):
    f32 = jnp.float32
    for b in range(BB):
        mb = mem_ref[b].astype(jnp.bfloat16)
        sq = mb * mb
        om = lax.dot_general(Lm_ref[b], mb, _DNK,
                             preferred_element_type=f32)
        osq = lax.dot_general(Lsq_ref[b], sq, _DNK,
                              preferred_element_type=f32)
        for r, ref in enumerate((dkw_ref, dkr_ref, dekr_ref, da_ref, dae_ref)):
            ref[b:b + 1, :] = om[r:r + 1, :]
        for r, ref in enumerate((ssq_ref, ssqe_ref, ssqee_ref)):
            ref[b:b + 1, :] = osq[r:r + 1, :]


def _run_pass1(mem_t, Lm, Lsq):
    big = pl.BlockSpec((BB, D, NB), lambda i, j: (i, 0, j))
    lspec = pl.BlockSpec((BB, 8, D), lambda i, j: (i, 0, 0))
    ospec = pl.BlockSpec((BB, NB), lambda i, j: (i, j))
    oshape = jax.ShapeDtypeStruct((B, N), jnp.float32)
    return pl.pallas_call(
        _pass1_kernel,
        grid=(GB, GJ),
        in_specs=[big, lspec, lspec],
        out_specs=[ospec] * 8,
        out_shape=[oshape] * 8,
        compiler_params=pltpu.CompilerParams(
            dimension_semantics=("parallel", "arbitrary")),
    )(mem_t, Lm, Lsq)


def _address(dots, sqn, par, pw):
    beta = par[:, 0:1]
    g = par[:, 1:2]
    gamma = par[:, 2:3]
    s0 = par[:, 3:4]
    s1 = par[:, 4:5]
    s2 = par[:, 5:6]
    knorm = jnp.sqrt(par[:, 6:7])
    norms = jnp.sqrt(jnp.maximum(sqn, 0.0)) * knorm
    z = beta * (dots / (norms + EPS))
    zmax = jnp.max(z, axis=-1, keepdims=True)
    ez = jnp.exp(z - zmax)
    wc = ez / jnp.sum(ez, axis=-1, keepdims=True)
    wg = g * wc + (1.0 - g) * pw
    roll_m1 = jnp.concatenate([wg[:, 1:], wg[:, :1]], axis=-1)
    roll_p1 = jnp.concatenate([wg[:, -1:], wg[:, :-1]], axis=-1)
    ws = s0 * roll_m1 + s1 * wg + s2 * roll_p1
    u = jnp.exp(gamma * jnp.log(ws))
    return u / (jnp.sum(u, axis=-1, keepdims=True) + EPS)


def _fin_w_kernel(dkw_ref, ssq_ref, par_ref, pw_ref, ww_ref):
    ww_ref[...] = _address(dkw_ref[...], ssq_ref[...], par_ref[...],
                           pw_ref[...])


def _run_fin_w(dkw, ssq, par_w, pw):
    row = pl.BlockSpec((BB, N), lambda i: (i, 0))
    return pl.pallas_call(
        _fin_w_kernel,
        grid=(GB,),
        in_specs=[row, row, pl.BlockSpec((BB, 128), lambda i: (i, 0)), row],
        out_specs=row,
        out_shape=jax.ShapeDtypeStruct((B, N), jnp.float32),
        compiler_params=pltpu.CompilerParams(
            dimension_semantics=("parallel",)),
    )(dkw, ssq, par_w, pw)


def _fin_r_kernel(dkr_ref, dekr_ref, da_ref, dae_ref,
                  ssq_ref, ssqe_ref, ssqee_ref,
                  ww_ref, par_ref, pr_ref,
                  wr_ref, wrww_ref, swr_ref):
    par = par_ref[...]
    ak = par[:, 7:8]
    asq = par[:, 8:9]
    ww = ww_ref[...]
    wwsq = ww * ww
    dots = dkr_ref[...] - ww * dekr_ref[...] + ww * ak
    sqn = (ssq_ref[...] - 2.0 * ww * ssqe_ref[...] + wwsq * ssqee_ref[...]
           + 2.0 * ww * da_ref[...] - 2.0 * wwsq * dae_ref[...] + wwsq * asq)
    wr = _address(dots, sqn, par, pr_ref[...])
    wr_ref[...] = wr
    p = wr * ww
    wrww_ref[...] = p
    swr_ref[...] = jnp.sum(p, axis=-1, keepdims=True)


def _run_fin_r(dkr, dekr, da, dae, ssq, ssqe, ssqee, ww, par_r, pr):
    row = pl.BlockSpec((BB, N), lambda i: (i, 0))
    return pl.pallas_call(
        _fin_r_kernel,
        grid=(GB,),
        in_specs=[row] * 7 + [row,
                              pl.BlockSpec((BB, 128), lambda i: (i, 0)), row],
        out_specs=[row, row, pl.BlockSpec((BB, 1), lambda i: (i, 0))],
        out_shape=[jax.ShapeDtypeStruct((B, N), jnp.float32),
                   jax.ShapeDtypeStruct((B, N), jnp.float32),
                   jax.ShapeDtypeStruct((B, 1), jnp.float32)],
        compiler_params=pltpu.CompilerParams(
            dimension_semantics=("parallel",)),
    )(dkr, dekr, da, dae, ssq, ssqe, ssqee, ww, par_r, pr)


def _pass2_kernel(mem_ref, wr_ref, wrww_ref, erase_ref, add_ref, swr_ref,
                  out_ref, acc1_ref, acc2_ref):
    j = pl.program_id(1)

    @pl.when(j == 0)
    def _init():
        acc1_ref[...] = jnp.zeros_like(acc1_ref)
        acc2_ref[...] = jnp.zeros_like(acc2_ref)

    for b in range(BB):
        mb = mem_ref[b]
        w1 = wr_ref[b:b + 1, :]
        w2 = wrww_ref[b:b + 1, :]
        p1 = mb * w1
        p2 = mb * w2
        f1 = sum([p1[:, k * 128:(k + 1) * 128] for k in range(NB // 128)])
        f2 = sum([p2[:, k * 128:(k + 1) * 128] for k in range(NB // 128)])
        acc1_ref[b] += f1
        acc2_ref[b] += f2

    @pl.when(j == GJ - 1)
    def _fin():
        cols1 = [jnp.sum(acc1_ref[b], axis=-1, keepdims=True)
                 for b in range(BB)]
        cols2 = [jnp.sum(acc2_ref[b], axis=-1, keepdims=True)
                 for b in range(BB)]
        p1t = jnp.concatenate(cols1, axis=-1).T
        p2t = jnp.concatenate(cols2, axis=-1).T
        out_ref[...] = (p1t - erase_ref[...] * p2t
                        + swr_ref[...] * add_ref[...])


def _run_pass2(mem_t, wr, wrww, erase, add, swr):
    big = pl.BlockSpec((BB, D, NB), lambda i, j: (i, 0, j))
    row = pl.BlockSpec((BB, NB), lambda i, j: (i, j))
    vec = pl.BlockSpec((BB, D), lambda i, j: (i, 0))
    return pl.pallas_call(
        _pass2_kernel,
        grid=(GB, GJ),
        in_specs=[big, row, row, vec, vec,
                  pl.BlockSpec((BB, 1), lambda i, j: (i, 0))],
        out_specs=vec,
        out_shape=jax.ShapeDtypeStruct((B, D), jnp.float32),
        scratch_shapes=[pltpu.VMEM((BB, D, 128), jnp.float32),
                        pltpu.VMEM((BB, D, 128), jnp.float32)],
        compiler_params=pltpu.CompilerParams(
            dimension_semantics=("parallel", "arbitrary")),
    )(mem_t, wr, wrww, erase, add, swr)


@jax.jit
def kernel(x, prev_memory, prev_read_weights, prev_write_weights,
           prev_read_vector,
           W_ctrl, b_ctrl,
           Wk_r, bk_r, Wb_r, bb_r, Wg_r, bg_r, Ws_r, bs_r, Wgam_r, bgam_r,
           Wk_w, bk_w, Wb_w, bb_w, Wg_w, bg_w, Ws_w, bs_w, Wgam_w, bgam_w,
           We_w, be_w, Wa_w, ba_w):
    ctrl_in = jnp.concatenate([x, prev_read_vector], axis=-1)

    def pack_scalar_weights(Wb, Wg, Wgam, Ws, bb, bg, bgam, bs):
        Wsc = jnp.concatenate([Wb, Wg, Wgam, Ws], axis=-1)
        Wsc = jnp.pad(Wsc, ((0, 0), (0, 122)))
        bsc = jnp.concatenate([bb, bg, bgam, bs], axis=-1)
        bsc = jnp.pad(bsc, (0, 122)).reshape(1, 128)
        return Wsc, bsc

    Wsc_r, bsc_r = pack_scalar_weights(Wb_r, Wg_r, Wgam_r, Ws_r,
                                       bb_r, bg_r, bgam_r, bs_r)
    Wsc_w, bsc_w = pack_scalar_weights(Wb_w, Wg_w, Wgam_w, Ws_w,
                                       bb_w, bg_w, bgam_w, bs_w)

    h, erase, add, Lm, Lsq, par_r, par_w = _run_prologue(
        ctrl_in, W_ctrl, b_ctrl.reshape(1, C),
        Wk_r, bk_r.reshape(1, D), Wk_w, bk_w.reshape(1, D),
        We_w, be_w.reshape(1, D), Wa_w, ba_w.reshape(1, D),
        Wsc_r, bsc_r, Wsc_w, bsc_w)

    mem_t = jnp.transpose(prev_memory, (0, 2, 1))

    dkw, dkr, dekr, da, dae, ssq, ssqe, ssqee = _run_pass1(mem_t, Lm, Lsq)
    ww = _run_fin_w(dkw, ssq, par_w, prev_write_weights)
    wr, wrww, swr = _run_fin_r(dkr, dekr, da, dae, ssq, ssqe, ssqee,
                               ww, par_r, prev_read_weights)
    read_vec = _run_pass2(mem_t, wr, wrww, erase, add, swr)
    return jnp.concatenate([h, read_vec], axis=-1)


# speedup vs baseline: 3.3034x; 1.2409x over previous
import jax
import jax.numpy as jnp
from jax import lax
from jax.experimental import pallas as pl
from jax.experimental.pallas import tpu as pltpu

B, N, D, C, IN, S = 64, 8192, 64, 256, 128, 3
CTRL_IN = IN + D
EPS = 1e-8

BB = 8
NB = 2048
GB = B // BB
GJ = N // NB

_DNK = (((1,), (0,)), ((), ()))


def _prologue_kernel(ctrl_in_ref, W_ctrl_ref, b_ctrl_ref,
                     Wk_r_ref, bk_r_ref, Wk_w_ref, bk_w_ref,
                     We_w_ref, be_w_ref, Wa_w_ref, ba_w_ref,
                     Wsc_r_ref, bsc_r_ref, Wsc_w_ref, bsc_w_ref,
                     h_ref, erase_ref, add_ref, Lm_ref, Lsq_ref,
                     par_r_ref, par_w_ref):
    f32 = jnp.float32
    h = jnp.maximum(
        jnp.dot(ctrl_in_ref[...], W_ctrl_ref[...],
                preferred_element_type=f32) + b_ctrl_ref[...], 0.0)
    h_ref[...] = h
    k_r = jnp.dot(h, Wk_r_ref[...], preferred_element_type=f32) + bk_r_ref[...]
    k_w = jnp.dot(h, Wk_w_ref[...], preferred_element_type=f32) + bk_w_ref[...]
    e = jax.nn.sigmoid(
        jnp.dot(h, We_w_ref[...], preferred_element_type=f32) + be_w_ref[...])
    a = jnp.tanh(
        jnp.dot(h, Wa_w_ref[...], preferred_element_type=f32) + ba_w_ref[...])
    erase_ref[...] = e
    add_ref[...] = a

    def pack_rows(rows):
        rs = [v[:, None, :] for v in rows]
        pad = jnp.zeros((B, 8 - len(rows), D), f32)
        return jnp.concatenate(rs + [pad], axis=1).astype(jnp.bfloat16)

    Lm_ref[...] = pack_rows([k_w, k_r, e * k_r, a, a * e])
    Lsq_ref[...] = pack_rows([jnp.ones((B, D), f32), e, e * e])

    ksq_r = jnp.sum(k_r * k_r, axis=-1, keepdims=True)
    ksq_w = jnp.sum(k_w * k_w, axis=-1, keepdims=True)
    ak = jnp.sum(a * k_r, axis=-1, keepdims=True)
    asq = jnp.sum(a * a, axis=-1, keepdims=True)
    for Wsc_ref, bsc_ref, ksq, extra, par_ref in (
            (Wsc_r_ref, bsc_r_ref, ksq_r, [ak, asq], par_r_ref),
            (Wsc_w_ref, bsc_w_ref, ksq_w, [], par_w_ref)):
        raw = (jnp.dot(h, Wsc_ref[...], preferred_element_type=f32)
               + bsc_ref[...])
        beta = jax.nn.softplus(raw[:, 0:1])
        g = jax.nn.sigmoid(raw[:, 1:2])
        gamma = jax.nn.softplus(raw[:, 2:3]) + 1.0
        slog = raw[:, 3:6]
        smax = jnp.max(slog, axis=-1, keepdims=True)
        sexp = jnp.exp(slog - smax)
        s = sexp / jnp.sum(sexp, axis=-1, keepdims=True)
        cols = [beta, g, gamma, s, ksq] + extra
        used = 7 + len(extra)
        cols.append(jnp.zeros((B, 128 - used), f32))
        par_ref[...] = jnp.concatenate(cols, axis=-1)


def _run_prologue(ctrl_in, W_ctrl, b_ctrl, Wk_r, bk_r, Wk_w, bk_w,
                  We_w, be_w, Wa_w, ba_w, Wsc_r, bsc_r, Wsc_w, bsc_w):
    out_shapes = (
        jax.ShapeDtypeStruct((B, C), jnp.float32),
        jax.ShapeDtypeStruct((B, D), jnp.float32),
        jax.ShapeDtypeStruct((B, D), jnp.float32),
        jax.ShapeDtypeStruct((B, 8, D), jnp.bfloat16),
        jax.ShapeDtypeStruct((B, 8, D), jnp.bfloat16),
        jax.ShapeDtypeStruct((B, 128), jnp.float32),
        jax.ShapeDtypeStruct((B, 128), jnp.float32),
    )
    return pl.pallas_call(
        _prologue_kernel,
        out_shape=out_shapes,
    )(ctrl_in, W_ctrl, b_ctrl, Wk_r, bk_r, Wk_w, bk_w,
      We_w, be_w, Wa_w, ba_w, Wsc_r, bsc_r, Wsc_w, bsc_w)


def _pass1_kernel(mem_ref, Lm_ref, Lsq_ref,
                  dkw_ref, dkr_ref, dekr_ref, da_ref, dae_ref,
                  ssq_ref, ssqe_ref, ssqee_ref):
    f32 = jnp.float32
    for b in range(BB):
        mb = mem_ref[b].astype(jnp.bfloat16)
        sq = mb * mb
        om = lax.dot_general(Lm_ref[b], mb, _DNK,
                             preferred_element_type=f32)
        osq = lax.dot_general(Lsq_ref[b], sq, _DNK,
                              preferred_element_type=f32)
        for r, ref in enumerate((dkw_ref, dkr_ref, dekr_ref, da_ref, dae_ref)):
            ref[b:b + 1, :] = om[r:r + 1, :]
        for r, ref in enumerate((ssq_ref, ssqe_ref, ssqee_ref)):
            ref[b:b + 1, :] = osq[r:r + 1, :]


def _run_pass1(mem_t, Lm, Lsq):
    big = pl.BlockSpec((BB, D, NB), lambda i, j: (i, 0, j))
    lspec = pl.BlockSpec((BB, 8, D), lambda i, j: (i, 0, 0))
    ospec = pl.BlockSpec((BB, NB), lambda i, j: (i, j))
    oshape = jax.ShapeDtypeStruct((B, N), jnp.float32)
    return pl.pallas_call(
        _pass1_kernel,
        grid=(GB, GJ),
        in_specs=[big, lspec, lspec],
        out_specs=[ospec] * 8,
        out_shape=[oshape] * 8,
        compiler_params=pltpu.CompilerParams(
            dimension_semantics=("parallel", "arbitrary")),
    )(mem_t, Lm, Lsq)


def _address(dots, sqn, par, pw):
    beta = par[:, 0:1]
    g = par[:, 1:2]
    gamma = par[:, 2:3]
    s0 = par[:, 3:4]
    s1 = par[:, 4:5]
    s2 = par[:, 5:6]
    knorm = jnp.sqrt(par[:, 6:7])
    norms = jnp.sqrt(jnp.maximum(sqn, 0.0)) * knorm
    z = beta * (dots / (norms + EPS))
    zmax = jnp.max(z, axis=-1, keepdims=True)
    ez = jnp.exp(z - zmax)
    wc = ez / jnp.sum(ez, axis=-1, keepdims=True)
    wg = g * wc + (1.0 - g) * pw
    roll_m1 = jnp.concatenate([wg[:, 1:], wg[:, :1]], axis=-1)
    roll_p1 = jnp.concatenate([wg[:, -1:], wg[:, :-1]], axis=-1)
    ws = s0 * roll_m1 + s1 * wg + s2 * roll_p1
    u = jnp.exp(gamma * jnp.log(ws))
    return u / (jnp.sum(u, axis=-1, keepdims=True) + EPS)


def _fin_w_kernel(dkw_ref, ssq_ref, par_ref, pw_ref, ww_ref):
    ww_ref[...] = _address(dkw_ref[...], ssq_ref[...], par_ref[...],
                           pw_ref[...])


def _run_fin_w(dkw, ssq, par_w, pw):
    row = pl.BlockSpec((BB, N), lambda i: (i, 0))
    return pl.pallas_call(
        _fin_w_kernel,
        grid=(GB,),
        in_specs=[row, row, pl.BlockSpec((BB, 128), lambda i: (i, 0)), row],
        out_specs=row,
        out_shape=jax.ShapeDtypeStruct((B, N), jnp.float32),
        compiler_params=pltpu.CompilerParams(
            dimension_semantics=("parallel",)),
    )(dkw, ssq, par_w, pw)


def _fin_r_kernel(dkr_ref, dekr_ref, da_ref, dae_ref,
                  ssq_ref, ssqe_ref, ssqee_ref,
                  ww_ref, par_ref, pr_ref,
                  wr_ref, wrww_ref, swr_ref):
    par = par_ref[...]
    ak = par[:, 7:8]
    asq = par[:, 8:9]
    ww = ww_ref[...]
    wwsq = ww * ww
    dots = dkr_ref[...] - ww * dekr_ref[...] + ww * ak
    sqn = (ssq_ref[...] - 2.0 * ww * ssqe_ref[...] + wwsq * ssqee_ref[...]
           + 2.0 * ww * da_ref[...] - 2.0 * wwsq * dae_ref[...] + wwsq * asq)
    wr = _address(dots, sqn, par, pr_ref[...])
    wr_ref[...] = wr
    p = wr * ww
    wrww_ref[...] = p
    swr_ref[...] = jnp.sum(p, axis=-1, keepdims=True)


def _run_fin_r(dkr, dekr, da, dae, ssq, ssqe, ssqee, ww, par_r, pr):
    row = pl.BlockSpec((BB, N), lambda i: (i, 0))
    return pl.pallas_call(
        _fin_r_kernel,
        grid=(GB,),
        in_specs=[row] * 7 + [row,
                              pl.BlockSpec((BB, 128), lambda i: (i, 0)), row],
        out_specs=[row, row, pl.BlockSpec((BB, 1), lambda i: (i, 0))],
        out_shape=[jax.ShapeDtypeStruct((B, N), jnp.float32),
                   jax.ShapeDtypeStruct((B, N), jnp.float32),
                   jax.ShapeDtypeStruct((B, 1), jnp.float32)],
        compiler_params=pltpu.CompilerParams(
            dimension_semantics=("parallel",)),
    )(dkr, dekr, da, dae, ssq, ssqe, ssqee, ww, par_r, pr)


def _pass2_kernel(mem_ref, wr_ref, wrww_ref, erase_ref, add_ref, swr_ref,
                  out_ref, acc1_ref, acc2_ref):
    j = pl.program_id(1)

    @pl.when(j == 0)
    def _init():
        acc1_ref[...] = jnp.zeros_like(acc1_ref)
        acc2_ref[...] = jnp.zeros_like(acc2_ref)

    for b in range(BB):
        mb = mem_ref[b]
        w1 = wr_ref[b:b + 1, :]
        w2 = wrww_ref[b:b + 1, :]
        p1 = mb * w1
        p2 = mb * w2
        f1 = sum([p1[:, k * 128:(k + 1) * 128] for k in range(NB // 128)])
        f2 = sum([p2[:, k * 128:(k + 1) * 128] for k in range(NB // 128)])
        acc1_ref[b] += f1
        acc2_ref[b] += f2

    @pl.when(j == GJ - 1)
    def _fin():
        cols1 = [jnp.sum(acc1_ref[b], axis=-1, keepdims=True)
                 for b in range(BB)]
        cols2 = [jnp.sum(acc2_ref[b], axis=-1, keepdims=True)
                 for b in range(BB)]
        p1t = jnp.concatenate(cols1, axis=-1).T
        p2t = jnp.concatenate(cols2, axis=-1).T
        out_ref[...] = (p1t - erase_ref[...] * p2t
                        + swr_ref[...] * add_ref[...])


def _run_pass2(mem_t, wr, wrww, erase, add, swr):
    big = pl.BlockSpec((BB, D, NB), lambda i, j: (i, 0, j))
    row = pl.BlockSpec((BB, NB), lambda i, j: (i, j))
    vec = pl.BlockSpec((BB, D), lambda i, j: (i, 0))
    return pl.pallas_call(
        _pass2_kernel,
        grid=(GB, GJ),
        in_specs=[big, row, row, vec, vec,
                  pl.BlockSpec((BB, 1), lambda i, j: (i, 0))],
        out_specs=vec,
        out_shape=jax.ShapeDtypeStruct((B, D), jnp.float32),
        scratch_shapes=[pltpu.VMEM((BB, D, 128), jnp.float32),
                        pltpu.VMEM((BB, D, 128), jnp.float32)],
        compiler_params=pltpu.CompilerParams(
            dimension_semantics=("parallel", "arbitrary")),
    )(mem_t, wr, wrww, erase, add, swr)


@jax.jit
def kernel(x, prev_memory, prev_read_weights, prev_write_weights,
           prev_read_vector,
           W_ctrl, b_ctrl,
           Wk_r, bk_r, Wb_r, bb_r, Wg_r, bg_r, Ws_r, bs_r, Wgam_r, bgam_r,
           Wk_w, bk_w, Wb_w, bb_w, Wg_w, bg_w, Ws_w, bs_w, Wgam_w, bgam_w,
           We_w, be_w, Wa_w, ba_w):
    ctrl_in = jnp.concatenate([x, prev_read_vector], axis=-1)

    def pack_scalar_weights(Wb, Wg, Wgam, Ws, bb, bg, bgam, bs):
        Wsc = jnp.concatenate([Wb, Wg, Wgam, Ws], axis=-1)
        Wsc = jnp.pad(Wsc, ((0, 0), (0, 122)))
        bsc = jnp.concatenate([bb, bg, bgam, bs], axis=-1)
        bsc = jnp.pad(bsc, (0, 122)).reshape(1, 128)
        return Wsc, bsc

    Wsc_r, bsc_r = pack_scalar_weights(Wb_r, Wg_r, Wgam_r, Ws_r,
                                       bb_r, bg_r, bgam_r, bs_r)
    Wsc_w, bsc_w = pack_scalar_weights(Wb_w, Wg_w, Wgam_w, Ws_w,
                                       bb_w, bg_w, bgam_w, bs_w)

    h, erase, add, Lm, Lsq, par_r, par_w = _run_prologue(
        ctrl_in, W_ctrl, b_ctrl.reshape(1, C),
        Wk_r, bk_r.reshape(1, D), Wk_w, bk_w.reshape(1, D),
        We_w, be_w.reshape(1, D), Wa_w, ba_w.reshape(1, D),
        Wsc_r, bsc_r, Wsc_w, bsc_w)

    mem_t = jnp.transpose(prev_memory, (0, 2, 1))

    dkw, dkr, dekr, da, dae, ssq, ssqe, ssqee = _run_pass1(mem_t, Lm, Lsq)
    ww = _run_fin_w(dkw, ssq, par_w, prev_write_weights)
    wr, wrww, swr = _run_fin_r(dkr, dekr, da, dae, ssq, ssqe, ssqee,
                               ww, par_r, prev_read_weights)
    read_vec = _run_pass2(mem_t, wr, wrww, erase, add, swr)
    return jnp.concatenate([h, read_vec], axis=-1)


# speedup vs baseline: 3.6530x; 1.1058x over previous
import jax
import jax.numpy as jnp
from jax import lax
from jax.experimental import pallas as pl
from jax.experimental.pallas import tpu as pltpu

B, N, D, C, IN, S = 64, 8192, 64, 256, 128, 3
CTRL_IN = IN + D
EPS = 1e-8

BB = 8
NB = 4096
GB = B // BB
GJ = N // NB

_DNK = (((1,), (0,)), ((), ()))


def _prologue_kernel(ctrl_in_ref, W_ctrl_ref, b_ctrl_ref,
                     Wk_r_ref, bk_r_ref, Wk_w_ref, bk_w_ref,
                     We_w_ref, be_w_ref, Wa_w_ref, ba_w_ref,
                     Wsc_r_ref, bsc_r_ref, Wsc_w_ref, bsc_w_ref,
                     h_ref, erase_ref, add_ref, Lm_ref, Lsq_ref,
                     par_r_ref, par_w_ref):
    f32 = jnp.float32
    h = jnp.maximum(
        jnp.dot(ctrl_in_ref[...], W_ctrl_ref[...],
                preferred_element_type=f32) + b_ctrl_ref[...], 0.0)
    h_ref[...] = h
    k_r = jnp.dot(h, Wk_r_ref[...], preferred_element_type=f32) + bk_r_ref[...]
    k_w = jnp.dot(h, Wk_w_ref[...], preferred_element_type=f32) + bk_w_ref[...]
    e = jax.nn.sigmoid(
        jnp.dot(h, We_w_ref[...], preferred_element_type=f32) + be_w_ref[...])
    a = jnp.tanh(
        jnp.dot(h, Wa_w_ref[...], preferred_element_type=f32) + ba_w_ref[...])
    erase_ref[...] = e
    add_ref[...] = a

    def pack_rows(rows):
        rs = [v[:, None, :] for v in rows]
        pad = jnp.zeros((B, 8 - len(rows), D), f32)
        return jnp.concatenate(rs + [pad], axis=1).astype(jnp.bfloat16)

    Lm_ref[...] = pack_rows([k_w, k_r, e * k_r, a, a * e])
    Lsq_ref[...] = pack_rows([jnp.ones((B, D), f32), e, e * e])

    ksq_r = jnp.sum(k_r * k_r, axis=-1, keepdims=True)
    ksq_w = jnp.sum(k_w * k_w, axis=-1, keepdims=True)
    ak = jnp.sum(a * k_r, axis=-1, keepdims=True)
    asq = jnp.sum(a * a, axis=-1, keepdims=True)
    for Wsc_ref, bsc_ref, ksq, extra, par_ref in (
            (Wsc_r_ref, bsc_r_ref, ksq_r, [ak, asq], par_r_ref),
            (Wsc_w_ref, bsc_w_ref, ksq_w, [], par_w_ref)):
        raw = (jnp.dot(h, Wsc_ref[...], preferred_element_type=f32)
               + bsc_ref[...])
        beta = jax.nn.softplus(raw[:, 0:1])
        g = jax.nn.sigmoid(raw[:, 1:2])
        gamma = jax.nn.softplus(raw[:, 2:3]) + 1.0
        slog = raw[:, 3:6]
        smax = jnp.max(slog, axis=-1, keepdims=True)
        sexp = jnp.exp(slog - smax)
        s = sexp / jnp.sum(sexp, axis=-1, keepdims=True)
        cols = [beta, g, gamma, s, ksq] + extra
        used = 7 + len(extra)
        cols.append(jnp.zeros((B, 128 - used), f32))
        par_ref[...] = jnp.concatenate(cols, axis=-1)


def _run_prologue(ctrl_in, W_ctrl, b_ctrl, Wk_r, bk_r, Wk_w, bk_w,
                  We_w, be_w, Wa_w, ba_w, Wsc_r, bsc_r, Wsc_w, bsc_w):
    out_shapes = (
        jax.ShapeDtypeStruct((B, C), jnp.float32),
        jax.ShapeDtypeStruct((B, D), jnp.float32),
        jax.ShapeDtypeStruct((B, D), jnp.float32),
        jax.ShapeDtypeStruct((B, 8, D), jnp.bfloat16),
        jax.ShapeDtypeStruct((B, 8, D), jnp.bfloat16),
        jax.ShapeDtypeStruct((B, 128), jnp.float32),
        jax.ShapeDtypeStruct((B, 128), jnp.float32),
    )
    return pl.pallas_call(
        _prologue_kernel,
        out_shape=out_shapes,
    )(ctrl_in, W_ctrl, b_ctrl, Wk_r, bk_r, Wk_w, bk_w,
      We_w, be_w, Wa_w, ba_w, Wsc_r, bsc_r, Wsc_w, bsc_w)


def _pass1_kernel(mem_ref, Lm_ref, Lsq_ref,
                  dkw_ref, dkr_ref, dekr_ref, da_ref, dae_ref,
                  ssq_ref, ssqe_ref, ssqee_ref):
    f32 = jnp.float32
    for b in range(BB):
        mb = mem_ref[b].astype(jnp.bfloat16)
        sq = mb * mb
        om = lax.dot_general(Lm_ref[b], mb, _DNK,
                             preferred_element_type=f32)
        osq = lax.dot_general(Lsq_ref[b], sq, _DNK,
                              preferred_element_type=f32)
        for r, ref in enumerate((dkw_ref, dkr_ref, dekr_ref, da_ref, dae_ref)):
            ref[b:b + 1, :] = om[r:r + 1, :]
        for r, ref in enumerate((ssq_ref, ssqe_ref, ssqee_ref)):
            ref[b:b + 1, :] = osq[r:r + 1, :]


def _run_pass1(mem_t, Lm, Lsq):
    big = pl.BlockSpec((BB, D, NB), lambda i, j: (i, 0, j))
    lspec = pl.BlockSpec((BB, 8, D), lambda i, j: (i, 0, 0))
    ospec = pl.BlockSpec((BB, NB), lambda i, j: (i, j))
    oshape = jax.ShapeDtypeStruct((B, N), jnp.float32)
    return pl.pallas_call(
        _pass1_kernel,
        grid=(GB, GJ),
        in_specs=[big, lspec, lspec],
        out_specs=[ospec] * 8,
        out_shape=[oshape] * 8,
        compiler_params=pltpu.CompilerParams(
            dimension_semantics=("parallel", "arbitrary")),
    )(mem_t, Lm, Lsq)


def _address(dots, sqn, par, pw):
    beta = par[:, 0:1]
    g = par[:, 1:2]
    gamma = par[:, 2:3]
    s0 = par[:, 3:4]
    s1 = par[:, 4:5]
    s2 = par[:, 5:6]
    knorm = jnp.sqrt(par[:, 6:7])
    norms = jnp.sqrt(jnp.maximum(sqn, 0.0)) * knorm
    z = beta * (dots / (norms + EPS))
    zmax = jnp.max(z, axis=-1, keepdims=True)
    ez = jnp.exp(z - zmax)
    wc = ez / jnp.sum(ez, axis=-1, keepdims=True)
    wg = g * wc + (1.0 - g) * pw
    roll_m1 = jnp.concatenate([wg[:, 1:], wg[:, :1]], axis=-1)
    roll_p1 = jnp.concatenate([wg[:, -1:], wg[:, :-1]], axis=-1)
    ws = s0 * roll_m1 + s1 * wg + s2 * roll_p1
    u = jnp.exp(gamma * jnp.log(ws))
    return u / (jnp.sum(u, axis=-1, keepdims=True) + EPS)


def _fin_w_kernel(dkw_ref, ssq_ref, par_ref, pw_ref, ww_ref):
    ww_ref[...] = _address(dkw_ref[...], ssq_ref[...], par_ref[...],
                           pw_ref[...])


def _run_fin_w(dkw, ssq, par_w, pw):
    row = pl.BlockSpec((BB, N), lambda i: (i, 0))
    return pl.pallas_call(
        _fin_w_kernel,
        grid=(GB,),
        in_specs=[row, row, pl.BlockSpec((BB, 128), lambda i: (i, 0)), row],
        out_specs=row,
        out_shape=jax.ShapeDtypeStruct((B, N), jnp.float32),
        compiler_params=pltpu.CompilerParams(
            dimension_semantics=("parallel",)),
    )(dkw, ssq, par_w, pw)


def _fin_r_kernel(dkr_ref, dekr_ref, da_ref, dae_ref,
                  ssq_ref, ssqe_ref, ssqee_ref,
                  ww_ref, par_ref, pr_ref,
                  wr_ref, wrww_ref, swr_ref):
    par = par_ref[...]
    ak = par[:, 7:8]
    asq = par[:, 8:9]
    ww = ww_ref[...]
    wwsq = ww * ww
    dots = dkr_ref[...] - ww * dekr_ref[...] + ww * ak
    sqn = (ssq_ref[...] - 2.0 * ww * ssqe_ref[...] + wwsq * ssqee_ref[...]
           + 2.0 * ww * da_ref[...] - 2.0 * wwsq * dae_ref[...] + wwsq * asq)
    wr = _address(dots, sqn, par, pr_ref[...])
    wr_ref[...] = wr
    p = wr * ww
    wrww_ref[...] = p
    swr_ref[...] = jnp.sum(p, axis=-1, keepdims=True)


def _run_fin_r(dkr, dekr, da, dae, ssq, ssqe, ssqee, ww, par_r, pr):
    row = pl.BlockSpec((BB, N), lambda i: (i, 0))
    return pl.pallas_call(
        _fin_r_kernel,
        grid=(GB,),
        in_specs=[row] * 7 + [row,
                              pl.BlockSpec((BB, 128), lambda i: (i, 0)), row],
        out_specs=[row, row, pl.BlockSpec((BB, 1), lambda i: (i, 0))],
        out_shape=[jax.ShapeDtypeStruct((B, N), jnp.float32),
                   jax.ShapeDtypeStruct((B, N), jnp.float32),
                   jax.ShapeDtypeStruct((B, 1), jnp.float32)],
        compiler_params=pltpu.CompilerParams(
            dimension_semantics=("parallel",)),
    )(dkr, dekr, da, dae, ssq, ssqe, ssqee, ww, par_r, pr)


def _pass2_kernel(mem_ref, wr_ref, wrww_ref, erase_ref, add_ref, swr_ref,
                  out_ref, acc1_ref, acc2_ref):
    j = pl.program_id(1)

    @pl.when(j == 0)
    def _init():
        acc1_ref[...] = jnp.zeros_like(acc1_ref)
        acc2_ref[...] = jnp.zeros_like(acc2_ref)

    for b in range(BB):
        mb = mem_ref[b]
        w1 = wr_ref[b:b + 1, :]
        w2 = wrww_ref[b:b + 1, :]
        p1 = mb * w1
        p2 = mb * w2
        f1 = sum([p1[:, k * 128:(k + 1) * 128] for k in range(NB // 128)])
        f2 = sum([p2[:, k * 128:(k + 1) * 128] for k in range(NB // 128)])
        acc1_ref[b] += f1
        acc2_ref[b] += f2

    @pl.when(j == GJ - 1)
    def _fin():
        cols1 = [jnp.sum(acc1_ref[b], axis=-1, keepdims=True)
                 for b in range(BB)]
        cols2 = [jnp.sum(acc2_ref[b], axis=-1, keepdims=True)
                 for b in range(BB)]
        p1t = jnp.concatenate(cols1, axis=-1).T
        p2t = jnp.concatenate(cols2, axis=-1).T
        out_ref[...] = (p1t - erase_ref[...] * p2t
                        + swr_ref[...] * add_ref[...])


def _run_pass2(mem_t, wr, wrww, erase, add, swr):
    big = pl.BlockSpec((BB, D, NB), lambda i, j: (i, 0, j))
    row = pl.BlockSpec((BB, NB), lambda i, j: (i, j))
    vec = pl.BlockSpec((BB, D), lambda i, j: (i, 0))
    return pl.pallas_call(
        _pass2_kernel,
        grid=(GB, GJ),
        in_specs=[big, row, row, vec, vec,
                  pl.BlockSpec((BB, 1), lambda i, j: (i, 0))],
        out_specs=vec,
        out_shape=jax.ShapeDtypeStruct((B, D), jnp.float32),
        scratch_shapes=[pltpu.VMEM((BB, D, 128), jnp.float32),
                        pltpu.VMEM((BB, D, 128), jnp.float32)],
        compiler_params=pltpu.CompilerParams(
            dimension_semantics=("parallel", "arbitrary")),
    )(mem_t, wr, wrww, erase, add, swr)


@jax.jit
def kernel(x, prev_memory, prev_read_weights, prev_write_weights,
           prev_read_vector,
           W_ctrl, b_ctrl,
           Wk_r, bk_r, Wb_r, bb_r, Wg_r, bg_r, Ws_r, bs_r, Wgam_r, bgam_r,
           Wk_w, bk_w, Wb_w, bb_w, Wg_w, bg_w, Ws_w, bs_w, Wgam_w, bgam_w,
           We_w, be_w, Wa_w, ba_w):
    ctrl_in = jnp.concatenate([x, prev_read_vector], axis=-1)

    def pack_scalar_weights(Wb, Wg, Wgam, Ws, bb, bg, bgam, bs):
        Wsc = jnp.concatenate([Wb, Wg, Wgam, Ws], axis=-1)
        Wsc = jnp.pad(Wsc, ((0, 0), (0, 122)))
        bsc = jnp.concatenate([bb, bg, bgam, bs], axis=-1)
        bsc = jnp.pad(bsc, (0, 122)).reshape(1, 128)
        return Wsc, bsc

    Wsc_r, bsc_r = pack_scalar_weights(Wb_r, Wg_r, Wgam_r, Ws_r,
                                       bb_r, bg_r, bgam_r, bs_r)
    Wsc_w, bsc_w = pack_scalar_weights(Wb_w, Wg_w, Wgam_w, Ws_w,
                                       bb_w, bg_w, bgam_w, bs_w)

    h, erase, add, Lm, Lsq, par_r, par_w = _run_prologue(
        ctrl_in, W_ctrl, b_ctrl.reshape(1, C),
        Wk_r, bk_r.reshape(1, D), Wk_w, bk_w.reshape(1, D),
        We_w, be_w.reshape(1, D), Wa_w, ba_w.reshape(1, D),
        Wsc_r, bsc_r, Wsc_w, bsc_w)

    mem_t = jnp.transpose(prev_memory, (0, 2, 1))

    dkw, dkr, dekr, da, dae, ssq, ssqe, ssqee = _run_pass1(mem_t, Lm, Lsq)
    ww = _run_fin_w(dkw, ssq, par_w, prev_write_weights)
    wr, wrww, swr = _run_fin_r(dkr, dekr, da, dae, ssq, ssqe, ssqee,
                               ww, par_r, prev_read_weights)
    read_vec = _run_pass2(mem_t, wr, wrww, erase, add, swr)
    return jnp.concatenate([h, read_vec], axis=-1)
